# Initial kernel scaffold; baseline (speedup 1.0000x reference)
import jax
import jax.numpy as jnp
from jax.experimental import pallas as pl


def kernel(positions, atomic_numbers, neighbors, emb, Wn, bn, Wgn, bgn, We, be, Wo1, bo1, Wo2, bo2):
    raise NotImplementedError("write your pallas kernel here")


# speedup vs baseline: 10.3653x; 10.3653x over previous
import functools

import jax
import jax.numpy as jnp
from jax import lax
from jax.experimental import pallas as pl
from jax.experimental.pallas import tpu as pltpu
from jax.experimental.pallas import tpu_sc as plsc

AT = 10000
ATP = 10240
NBR = 32
F = 128
FE = 128
GAUSS_END = 8.0
BA = 128
NBLK = ATP // BA
E = ATP * NBR

SC_NC = 2
SC_NS = 16
SC_NW = SC_NC * SC_NS
EPW = E // SC_NW
CH = 128
NCHUNK = EPW // CH
L = 16


def _sc_gather_rows(table, idx):
    V, D = table.shape
    mesh = plsc.VectorSubcoreMesh(core_axis_name="c", subcore_axis_name="s")

    @functools.partial(
        pl.kernel,
        out_type=jax.ShapeDtypeStruct((E, D), jnp.float32),
        mesh=mesh,
        scratch_types=[
            pltpu.VMEM((EPW,), jnp.int32),
            pltpu.VMEM((4, CH, D), jnp.float32),
            pltpu.SemaphoreType.DMA,
            pltpu.SemaphoreType.DMA,
            pltpu.SemaphoreType.DMA,
            pltpu.SemaphoreType.DMA,
            pltpu.SemaphoreType.DMA,
            pltpu.SemaphoreType.DMA,
            pltpu.SemaphoreType.DMA,
            pltpu.SemaphoreType.DMA,
        ],
    )
    def gather_k(table_hbm, idx_hbm, out_hbm, idx_v, bufs,
                 g0, g1, g2, g3, w0, w1, w2, w3):
        gsem = (g0, g1, g2, g3)
        wsem = (w0, w1, w2, w3)
        wid = lax.axis_index("s") * SC_NC + lax.axis_index("c")
        base = pl.multiple_of(wid * EPW, 8)
        pltpu.sync_copy(idx_hbm.at[pl.ds(base, EPW)], idx_v)

        def gcopy(c, b):
            off = pl.multiple_of(c * CH, 8)
            return pltpu.make_async_copy(
                table_hbm.at[idx_v.at[pl.ds(off, CH)]], bufs.at[b], gsem[b])

        def wcopy(c, b):
            off = pl.multiple_of(base + c * CH, 8)
            return pltpu.make_async_copy(
                bufs.at[b], out_hbm.at[pl.ds(off, CH)], wsem[b])

        gcopy(0, 0).start()
        gcopy(1, 1).start()

        def body(p, carry):
            c0 = 4 * p
            for b in range(4):
                c = c0 + b
                gcopy(c, b).wait()
                wcopy(c, b).start()

                @pl.when(c + 2 < NCHUNK)
                def _():
                    @pl.when(c >= 2)
                    def _():
                        wcopy(c - 2, (b - 2) % 4).wait()
                    gcopy(c + 2, (b + 2) % 4).start()
            return carry

        lax.fori_loop(0, NCHUNK // 4, body, 0)
        for c in range(NCHUNK - 4, NCHUNK):
            wcopy(c, c % 4).wait()

    return gather_k(table, idx)


def _sc_gather_coords(px, py, pz, an, idx):
    mesh = plsc.VectorSubcoreMesh(core_axis_name="c", subcore_axis_name="s")
    shp = jax.ShapeDtypeStruct((E,), jnp.float32)
    shpi = jax.ShapeDtypeStruct((E,), jnp.int32)

    @functools.partial(
        pl.kernel,
        out_type=[shp, shp, shp, shpi],
        mesh=mesh,
        compiler_params=pltpu.CompilerParams(needs_layout_passes=False),
        scratch_types=[
            pltpu.VMEM((ATP,), jnp.float32),
            pltpu.VMEM((ATP,), jnp.float32),
            pltpu.VMEM((ATP,), jnp.float32),
            pltpu.VMEM((ATP,), jnp.int32),
            pltpu.VMEM((EPW,), jnp.int32),
            pltpu.VMEM((EPW,), jnp.float32),
            pltpu.VMEM((EPW,), jnp.float32),
            pltpu.VMEM((EPW,), jnp.float32),
            pltpu.VMEM((EPW,), jnp.int32),
        ],
    )
    def coords_k(px_hbm, py_hbm, pz_hbm, an_hbm, idx_hbm,
                 ox_hbm, oy_hbm, oz_hbm, oa_hbm,
                 px_v, py_v, pz_v, an_v, idx_v, bx_v, by_v, bz_v, ba_v):
        wid = lax.axis_index("s") * SC_NC + lax.axis_index("c")
        base = pl.multiple_of(wid * EPW, 8)
        pltpu.sync_copy(px_hbm, px_v)
        pltpu.sync_copy(py_hbm, py_v)
        pltpu.sync_copy(pz_hbm, pz_v)
        pltpu.sync_copy(an_hbm, an_v)
        pltpu.sync_copy(idx_hbm.at[pl.ds(base, EPW)], idx_v)

        def body(g, carry):
            off = pl.multiple_of(g * L, 8)
            idxj = idx_v[pl.ds(off, L)]
            bx_v[pl.ds(off, L)] = plsc.load_gather(px_v, [idxj])
            by_v[pl.ds(off, L)] = plsc.load_gather(py_v, [idxj])
            bz_v[pl.ds(off, L)] = plsc.load_gather(pz_v, [idxj])
            ba_v[pl.ds(off, L)] = plsc.load_gather(an_v, [idxj])
            return carry

        lax.fori_loop(0, EPW // L, body, 0)
        pltpu.sync_copy(bx_v, ox_hbm.at[pl.ds(base, EPW)])
        pltpu.sync_copy(by_v, oy_hbm.at[pl.ds(base, EPW)])
        pltpu.sync_copy(bz_v, oz_hbm.at[pl.ds(base, EPW)])
        pltpu.sync_copy(ba_v, oa_hbm.at[pl.ds(base, EPW)])

    return coords_k(px, py, pz, an, idx)


def _geom_body(pos_ref, xj_ref, yj_ref, zj_ref, edge_ref, unit_ref):
    pos = pos_ref[...]
    rx = xj_ref[...] - pos[:, 0:1]
    ry = yj_ref[...] - pos[:, 1:2]
    rz = zj_ref[...] - pos[:, 2:3]
    dist = jnp.sqrt(rx * rx + ry * ry + rz * rz + 1e-10)
    inv = 1.0 / dist
    lane4 = lax.broadcasted_iota(jnp.int32, (BA, NBR, 4), 2)
    unit_ref[...] = jnp.where(
        lane4 == 0, (rx * inv)[:, :, None],
        jnp.where(lane4 == 1, (ry * inv)[:, :, None],
                  jnp.where(lane4 == 2, (rz * inv)[:, :, None], 0.0)))
    centers = lax.broadcasted_iota(jnp.int32, (BA, NBR, FE), 2).astype(
        jnp.float32) * (GAUSS_END / (FE - 1))
    z = (dist[:, :, None] - centers) * ((FE - 1) / GAUSS_END)
    edge_ref[...] = jnp.exp(-0.5 * z * z)


def _layer0_body(an_ref, anj_ref, edge_ref, emb_ref, wself_ref, wnbh_ref,
                 wedge_ref, bias_ref, node_o_ref, edge_o_ref):
    emb = emb_ref[...]
    an = an_ref[...]
    onehot_i = (an == lax.broadcasted_iota(jnp.int32, (BA, 128), 1)
                ).astype(jnp.float32)
    nodeb = jnp.dot(onehot_i, emb, preferred_element_type=jnp.float32)
    self_proj = jnp.dot(nodeb, wself_ref[...],
                        preferred_element_type=jnp.float32) + bias_ref[...]
    embw = jnp.dot(emb, wnbh_ref[...],
                   preferred_element_type=jnp.float32)
    anj3 = anj_ref[...][:, :, None]
    onehot_j = (anj3 == lax.broadcasted_iota(jnp.int32, (BA, NBR, 128), 2)
                ).astype(jnp.float32).reshape(BA * NBR, 128)
    edg = edge_ref[...].reshape(BA * NBR, FE)
    u = (jnp.dot(onehot_j, embw, preferred_element_type=jnp.float32)
         + jnp.dot(edg, wedge_ref[...], preferred_element_type=jnp.float32))
    u3 = u.reshape(BA, NBR, 3 * F) + self_proj[:, None, :]
    un = u3[..., 0:F]
    ug = u3[..., F:2 * F]
    ue = u3[..., 2 * F:3 * F]
    msg = jnp.tanh(un) * (1.0 / (1.0 + jnp.exp(-ug)))
    node_o_ref[...] = nodeb + jnp.sum(msg, axis=1)
    edge_o_ref[...] = edge_ref[...] + jnp.tanh(ue)


def _layer1_body(node_ref, nbh_ref, edge_ref, unit_ref, wself_ref, wnbh_ref,
                 wedge_ref, be_ref, wo1_ref, bo1_ref, wo2_ref, bo2_ref,
                 out_ref):
    self_proj = jnp.dot(node_ref[...], wself_ref[...],
                        preferred_element_type=jnp.float32) + be_ref[...]
    nbh = nbh_ref[...].reshape(BA * NBR, F)
    edg = edge_ref[...].reshape(BA * NBR, FE)
    u = (jnp.dot(nbh, wnbh_ref[...], preferred_element_type=jnp.float32)
         + jnp.dot(edg, wedge_ref[...], preferred_element_type=jnp.float32))
    u3 = u.reshape(BA, NBR, FE) + self_proj[:, None, :]
    e2 = edge_ref[...] + jnp.tanh(u3)
    h = jnp.dot(e2.reshape(BA * NBR, FE), wo1_ref[...],
                preferred_element_type=jnp.float32) + bo1_ref[...]
    hs = jnp.maximum(h, 0.0) + jnp.log(1.0 + jnp.exp(-jnp.abs(h)))
    fm = jnp.sum(hs.reshape(BA, NBR, FE // 2) * wo2_ref[...][:, None, :],
                 axis=-1) + bo2_ref[0]
    out_ref[...] = jnp.sum(fm[:, :, None] * unit_ref[...], axis=1)


def _full(shape):
    return pl.BlockSpec(shape, lambda i: tuple(0 for _ in shape))


def kernel(positions, atomic_numbers, neighbors, emb, Wn, bn, Wgn, bgn,
           We, be, Wo1, bo1, Wo2, bo2):
    pos = jnp.pad(positions[0], ((0, ATP - AT), (0, 0)))
    pos16 = jnp.pad(pos, ((0, 0), (0, 13)))
    an = jnp.pad(atomic_numbers[0].astype(jnp.int32),
                 (0, ATP - AT)).reshape(ATP, 1)
    nbr = jnp.pad(neighbors[0].astype(jnp.int32).reshape(AT * NBR),
                  (0, E - AT * NBR))
    emb_pad = jnp.pad(emb, ((0, 128 - emb.shape[0]), (0, 0)))

    def third(k):
        return jnp.concatenate(
            [Wn[0][k * F:(k + 1) * F], Wgn[0][k * F:(k + 1) * F],
             We[0][k * F:(k + 1) * F]], axis=1)
    wself0, wnbh0, wedge0 = third(0), third(1), third(2)
    bias0 = jnp.concatenate([bn[0], bgn[0], be[0]]).reshape(1, 3 * F)
    wself1 = We[1][0:F]
    wnbh1 = We[1][F:2 * F]
    wedge1 = We[1][2 * F:3 * F]
    bias1 = be[1].reshape(1, FE)

    xj, yj, zj, anj = _sc_gather_coords(
        pos[:, 0], pos[:, 1], pos[:, 2], an.reshape(ATP), nbr)
    xj = xj.reshape(ATP, NBR)
    yj = yj.reshape(ATP, NBR)
    zj = zj.reshape(ATP, NBR)
    anj = anj.reshape(ATP, NBR)

    edge0, unit = pl.pallas_call(
        _geom_body,
        grid=(NBLK,),
        in_specs=[
            pl.BlockSpec((BA, 16), lambda i: (i, 0)),
            pl.BlockSpec((BA, NBR), lambda i: (i, 0)),
            pl.BlockSpec((BA, NBR), lambda i: (i, 0)),
            pl.BlockSpec((BA, NBR), lambda i: (i, 0)),
        ],
        out_specs=[
            pl.BlockSpec((BA, NBR, FE), lambda i: (i, 0, 0)),
            pl.BlockSpec((BA, NBR, 4), lambda i: (i, 0, 0)),
        ],
        out_shape=[
            jax.ShapeDtypeStruct((ATP, NBR, FE), jnp.float32),
            jax.ShapeDtypeStruct((ATP, NBR, 4), jnp.float32),
        ],
    )(pos16, xj, yj, zj)

    node1, edge1 = pl.pallas_call(
        _layer0_body,
        grid=(NBLK,),
        in_specs=[
            pl.BlockSpec((BA, 1), lambda i: (i, 0)),
            pl.BlockSpec((BA, NBR), lambda i: (i, 0)),
            pl.BlockSpec((BA, NBR, FE), lambda i: (i, 0, 0)),
            _full((128, F)),
            _full((F, 3 * F)),
            _full((F, 3 * F)),
            _full((FE, 3 * F)),
            _full((1, 3 * F)),
        ],
        out_specs=[
            pl.BlockSpec((BA, F), lambda i: (i, 0)),
            pl.BlockSpec((BA, NBR, FE), lambda i: (i, 0, 0)),
        ],
        out_shape=[
            jax.ShapeDtypeStruct((ATP, F), jnp.float32),
            jax.ShapeDtypeStruct((ATP, NBR, FE), jnp.float32),
        ],
    )(an, anj, edge0, emb_pad, wself0, wnbh0, wedge0, bias0)

    nbh1 = _sc_gather_rows(node1, nbr).reshape(ATP, NBR, F)

    forces = pl.pallas_call(
        _layer1_body,
        grid=(NBLK,),
        in_specs=[
            pl.BlockSpec((BA, F), lambda i: (i, 0)),
            pl.BlockSpec((BA, NBR, F), lambda i: (i, 0, 0)),
            pl.BlockSpec((BA, NBR, FE), lambda i: (i, 0, 0)),
            pl.BlockSpec((BA, NBR, 4), lambda i: (i, 0, 0)),
            _full((F, FE)),
            _full((F, FE)),
            _full((FE, FE)),
            _full((1, FE)),
            _full((FE, FE // 2)),
            _full((1, FE // 2)),
            _full((1, FE // 2)),
            pl.BlockSpec(memory_space=pltpu.SMEM),
        ],
        out_specs=[pl.BlockSpec((BA, 4), lambda i: (i, 0))],
        out_shape=[jax.ShapeDtypeStruct((ATP, 4), jnp.float32)],
    )(node1, nbh1, edge1, unit, wself1, wnbh1, wedge1, bias1,
      Wo1, bo1.reshape(1, FE // 2), Wo2.reshape(1, FE // 2), bo2)[0]

    return forces[0:AT, 0:3].reshape(1, AT, 3)


# speedup vs baseline: 10.4209x; 1.0054x over previous
import functools

import jax
import jax.numpy as jnp
from jax import lax
from jax.experimental import pallas as pl
from jax.experimental.pallas import tpu as pltpu
from jax.experimental.pallas import tpu_sc as plsc

AT = 10000
ATP = 10240
NBR = 32
F = 128
FE = 128
GAUSS_END = 8.0
BA = 128
NBLK = ATP // BA
E = ATP * NBR

SC_NC = 2
SC_NS = 16
SC_NW = SC_NC * SC_NS
EPW = E // SC_NW
CH = 128
NCHUNK = EPW // CH
NBUF = 5
LK = 3
L = 16


def _sc_gather_rows(table, idx):
    V, D = table.shape
    mesh = plsc.VectorSubcoreMesh(core_axis_name="c", subcore_axis_name="s")

    @functools.partial(
        pl.kernel,
        out_type=jax.ShapeDtypeStruct((E, D), jnp.float32),
        mesh=mesh,
        scratch_types=[
            pltpu.VMEM((EPW,), jnp.int32),
            pltpu.VMEM((NBUF, CH, D), jnp.float32),
        ] + [pltpu.SemaphoreType.DMA] * (2 * NBUF),
    )
    def gather_k(table_hbm, idx_hbm, out_hbm, idx_v, bufs, *sems):
        gsem = sems[:NBUF]
        wsem = sems[NBUF:]
        wid = lax.axis_index("s") * SC_NC + lax.axis_index("c")
        base = pl.multiple_of(wid * EPW, 8)
        pltpu.sync_copy(idx_hbm.at[pl.ds(base, EPW)], idx_v)

        def gcopy(c, b):
            off = pl.multiple_of(c * CH, 8)
            return pltpu.make_async_copy(
                table_hbm.at[idx_v.at[pl.ds(off, CH)]], bufs.at[b], gsem[b])

        def wcopy(c, b):
            off = pl.multiple_of(base + c * CH, 8)
            return pltpu.make_async_copy(
                bufs.at[b], out_hbm.at[pl.ds(off, CH)], wsem[b])

        for c in range(LK):
            gcopy(c, c).start()

        def body(p, carry):
            c0 = NBUF * p
            for b in range(NBUF):
                c = c0 + b
                gcopy(c, b).wait()
                wcopy(c, b).start()

                @pl.when(c + LK < NCHUNK)
                def _():
                    @pl.when(c + LK >= NBUF)
                    def _():
                        wcopy(c + LK - NBUF, (b + LK) % NBUF).wait()
                    gcopy(c + LK, (b + LK) % NBUF).start()
            return carry

        lax.fori_loop(0, NCHUNK // NBUF, body, 0)
        for c in range(NCHUNK - NBUF, NCHUNK):
            wcopy(c, c % NBUF).wait()

    return gather_k(table, idx)


def _sc_gather_coords(px, py, pz, an, idx):
    mesh = plsc.VectorSubcoreMesh(core_axis_name="c", subcore_axis_name="s")
    shp = jax.ShapeDtypeStruct((E,), jnp.float32)
    shpi = jax.ShapeDtypeStruct((E,), jnp.int32)

    @functools.partial(
        pl.kernel,
        out_type=[shp, shp, shp, shpi],
        mesh=mesh,
        compiler_params=pltpu.CompilerParams(needs_layout_passes=False),
        scratch_types=[
            pltpu.VMEM((ATP,), jnp.float32),
            pltpu.VMEM((ATP,), jnp.float32),
            pltpu.VMEM((ATP,), jnp.float32),
            pltpu.VMEM((ATP,), jnp.int32),
            pltpu.VMEM((EPW,), jnp.int32),
            pltpu.VMEM((EPW,), jnp.float32),
            pltpu.VMEM((EPW,), jnp.float32),
            pltpu.VMEM((EPW,), jnp.float32),
            pltpu.VMEM((EPW,), jnp.int32),
        ],
    )
    def coords_k(px_hbm, py_hbm, pz_hbm, an_hbm, idx_hbm,
                 ox_hbm, oy_hbm, oz_hbm, oa_hbm,
                 px_v, py_v, pz_v, an_v, idx_v, bx_v, by_v, bz_v, ba_v):
        wid = lax.axis_index("s") * SC_NC + lax.axis_index("c")
        base = pl.multiple_of(wid * EPW, 8)
        pltpu.sync_copy(px_hbm, px_v)
        pltpu.sync_copy(py_hbm, py_v)
        pltpu.sync_copy(pz_hbm, pz_v)
        pltpu.sync_copy(an_hbm, an_v)
        pltpu.sync_copy(idx_hbm.at[pl.ds(base, EPW)], idx_v)

        def body(g, carry):
            off = pl.multiple_of(g * L, 8)
            idxj = idx_v[pl.ds(off, L)]
            bx_v[pl.ds(off, L)] = plsc.load_gather(px_v, [idxj])
            by_v[pl.ds(off, L)] = plsc.load_gather(py_v, [idxj])
            bz_v[pl.ds(off, L)] = plsc.load_gather(pz_v, [idxj])
            ba_v[pl.ds(off, L)] = plsc.load_gather(an_v, [idxj])
            return carry

        lax.fori_loop(0, EPW // L, body, 0)
        pltpu.sync_copy(bx_v, ox_hbm.at[pl.ds(base, EPW)])
        pltpu.sync_copy(by_v, oy_hbm.at[pl.ds(base, EPW)])
        pltpu.sync_copy(bz_v, oz_hbm.at[pl.ds(base, EPW)])
        pltpu.sync_copy(ba_v, oa_hbm.at[pl.ds(base, EPW)])

    return coords_k(px, py, pz, an, idx)


def _geom_body(pos_ref, xj_ref, yj_ref, zj_ref, edge_ref, unit_ref):
    pos = pos_ref[...]
    rx = xj_ref[...] - pos[:, 0:1]
    ry = yj_ref[...] - pos[:, 1:2]
    rz = zj_ref[...] - pos[:, 2:3]
    dist = jnp.sqrt(rx * rx + ry * ry + rz * rz + 1e-10)
    inv = 1.0 / dist
    lane4 = lax.broadcasted_iota(jnp.int32, (BA, NBR, 4), 2)
    unit_ref[...] = jnp.where(
        lane4 == 0, (rx * inv)[:, :, None],
        jnp.where(lane4 == 1, (ry * inv)[:, :, None],
                  jnp.where(lane4 == 2, (rz * inv)[:, :, None], 0.0)))
    centers = lax.broadcasted_iota(jnp.int32, (BA, NBR, FE), 2).astype(
        jnp.float32) * (GAUSS_END / (FE - 1))
    z = (dist[:, :, None] - centers) * ((FE - 1) / GAUSS_END)
    edge_ref[...] = jnp.exp(-0.5 * z * z)


def _layer0_body(an_ref, anj_ref, edge_ref, emb_ref, wself_ref, wnbh_ref,
                 wedge_ref, bias_ref, wnbh1_ref, node_o_ref, edge_o_ref,
                 proj_o_ref):
    emb = emb_ref[...]
    an = an_ref[...]
    onehot_i = (an == lax.broadcasted_iota(jnp.int32, (BA, 128), 1)
                ).astype(jnp.float32)
    nodeb = jnp.dot(onehot_i, emb, preferred_element_type=jnp.float32)
    self_proj = jnp.dot(nodeb, wself_ref[...],
                        preferred_element_type=jnp.float32) + bias_ref[...]
    embw = jnp.dot(emb, wnbh_ref[...],
                   preferred_element_type=jnp.float32)
    anj3 = anj_ref[...][:, :, None]
    onehot_j = (anj3 == lax.broadcasted_iota(jnp.int32, (BA, NBR, 128), 2)
                ).astype(jnp.float32).reshape(BA * NBR, 128)
    edg = edge_ref[...].reshape(BA * NBR, FE)
    u = (jnp.dot(onehot_j, embw, preferred_element_type=jnp.float32)
         + jnp.dot(edg, wedge_ref[...], preferred_element_type=jnp.float32))
    u3 = u.reshape(BA, NBR, 3 * F) + self_proj[:, None, :]
    un = u3[..., 0:F]
    ug = u3[..., F:2 * F]
    ue = u3[..., 2 * F:3 * F]
    msg = jnp.tanh(un) * (1.0 / (1.0 + jnp.exp(-ug)))
    node1 = nodeb + jnp.sum(msg, axis=1)
    node_o_ref[...] = node1
    edge_o_ref[...] = edge_ref[...] + jnp.tanh(ue)
    proj_o_ref[...] = jnp.dot(node1, wnbh1_ref[...],
                              preferred_element_type=jnp.float32)


def _layer1_body(node_ref, nbhp_ref, edge_ref, unit_ref, wself_ref,
                 wedge_ref, be_ref, wo1_ref, bo1_ref, wo2_ref, bo2_ref,
                 out_ref):
    self_proj = jnp.dot(node_ref[...], wself_ref[...],
                        preferred_element_type=jnp.float32) + be_ref[...]
    edg = edge_ref[...].reshape(BA * NBR, FE)
    u = (nbhp_ref[...].reshape(BA * NBR, FE)
         + jnp.dot(edg, wedge_ref[...], preferred_element_type=jnp.float32))
    u3 = u.reshape(BA, NBR, FE) + self_proj[:, None, :]
    e2 = edge_ref[...] + jnp.tanh(u3)
    h = jnp.dot(e2.reshape(BA * NBR, FE), wo1_ref[...],
                preferred_element_type=jnp.float32) + bo1_ref[...]
    hs = jnp.maximum(h, 0.0) + jnp.log(1.0 + jnp.exp(-jnp.abs(h)))
    fm = jnp.sum(hs.reshape(BA, NBR, FE // 2) * wo2_ref[...][:, None, :],
                 axis=-1) + bo2_ref[0]
    out_ref[...] = jnp.sum(fm[:, :, None] * unit_ref[...], axis=1)


def _full(shape):
    return pl.BlockSpec(shape, lambda i: tuple(0 for _ in shape))


def kernel(positions, atomic_numbers, neighbors, emb, Wn, bn, Wgn, bgn,
           We, be, Wo1, bo1, Wo2, bo2):
    pos = jnp.pad(positions[0], ((0, ATP - AT), (0, 0)))
    pos16 = jnp.pad(pos, ((0, 0), (0, 13)))
    an = jnp.pad(atomic_numbers[0].astype(jnp.int32),
                 (0, ATP - AT)).reshape(ATP, 1)
    nbr = jnp.pad(neighbors[0].astype(jnp.int32).reshape(AT * NBR),
                  (0, E - AT * NBR))
    emb_pad = jnp.pad(emb, ((0, 128 - emb.shape[0]), (0, 0)))

    def third(k):
        return jnp.concatenate(
            [Wn[0][k * F:(k + 1) * F], Wgn[0][k * F:(k + 1) * F],
             We[0][k * F:(k + 1) * F]], axis=1)
    wself0, wnbh0, wedge0 = third(0), third(1), third(2)
    bias0 = jnp.concatenate([bn[0], bgn[0], be[0]]).reshape(1, 3 * F)
    wself1 = We[1][0:F]
    wnbh1 = We[1][F:2 * F]
    wedge1 = We[1][2 * F:3 * F]
    bias1 = be[1].reshape(1, FE)

    xj, yj, zj, anj = _sc_gather_coords(
        pos[:, 0], pos[:, 1], pos[:, 2], an.reshape(ATP), nbr)
    xj = xj.reshape(ATP, NBR)
    yj = yj.reshape(ATP, NBR)
    zj = zj.reshape(ATP, NBR)
    anj = anj.reshape(ATP, NBR)

    edge0, unit = pl.pallas_call(
        _geom_body,
        grid=(NBLK,),
        in_specs=[
            pl.BlockSpec((BA, 16), lambda i: (i, 0)),
            pl.BlockSpec((BA, NBR), lambda i: (i, 0)),
            pl.BlockSpec((BA, NBR), lambda i: (i, 0)),
            pl.BlockSpec((BA, NBR), lambda i: (i, 0)),
        ],
        out_specs=[
            pl.BlockSpec((BA, NBR, FE), lambda i: (i, 0, 0)),
            pl.BlockSpec((BA, NBR, 4), lambda i: (i, 0, 0)),
        ],
        out_shape=[
            jax.ShapeDtypeStruct((ATP, NBR, FE), jnp.float32),
            jax.ShapeDtypeStruct((ATP, NBR, 4), jnp.float32),
        ],
    )(pos16, xj, yj, zj)

    node1, edge1, proj1 = pl.pallas_call(
        _layer0_body,
        grid=(NBLK,),
        in_specs=[
            pl.BlockSpec((BA, 1), lambda i: (i, 0)),
            pl.BlockSpec((BA, NBR), lambda i: (i, 0)),
            pl.BlockSpec((BA, NBR, FE), lambda i: (i, 0, 0)),
            _full((128, F)),
            _full((F, 3 * F)),
            _full((F, 3 * F)),
            _full((FE, 3 * F)),
            _full((1, 3 * F)),
            _full((F, FE)),
        ],
        out_specs=[
            pl.BlockSpec((BA, F), lambda i: (i, 0)),
            pl.BlockSpec((BA, NBR, FE), lambda i: (i, 0, 0)),
            pl.BlockSpec((BA, FE), lambda i: (i, 0)),
        ],
        out_shape=[
            jax.ShapeDtypeStruct((ATP, F), jnp.float32),
            jax.ShapeDtypeStruct((ATP, NBR, FE), jnp.float32),
            jax.ShapeDtypeStruct((ATP, FE), jnp.float32),
        ],
    )(an, anj, edge0, emb_pad, wself0, wnbh0, wedge0, bias0, wnbh1)

    nbhp1 = _sc_gather_rows(proj1, nbr).reshape(ATP, NBR, FE)

    forces = pl.pallas_call(
        _layer1_body,
        grid=(NBLK,),
        in_specs=[
            pl.BlockSpec((BA, F), lambda i: (i, 0)),
            pl.BlockSpec((BA, NBR, FE), lambda i: (i, 0, 0)),
            pl.BlockSpec((BA, NBR, FE), lambda i: (i, 0, 0)),
            pl.BlockSpec((BA, NBR, 4), lambda i: (i, 0, 0)),
            _full((F, FE)),
            _full((FE, FE)),
            _full((1, FE)),
            _full((FE, FE // 2)),
            _full((1, FE // 2)),
            _full((1, FE // 2)),
            pl.BlockSpec(memory_space=pltpu.SMEM),
        ],
        out_specs=[pl.BlockSpec((BA, 4), lambda i: (i, 0))],
        out_shape=[jax.ShapeDtypeStruct((ATP, 4), jnp.float32)],
    )(node1, nbhp1, edge1, unit, wself1, wedge1, bias1,
      Wo1, bo1.reshape(1, FE // 2), Wo2.reshape(1, FE // 2), bo2)[0]

    return forces[0:AT, 0:3].reshape(1, AT, 3)


# speedup vs baseline: 15.7727x; 1.5136x over previous
import functools

import jax
import jax.numpy as jnp
from jax import lax
from jax.experimental import pallas as pl
from jax.experimental.pallas import tpu as pltpu
from jax.experimental.pallas import tpu_sc as plsc

AT = 10000
ATP = 10240
NBR = 32
F = 128
FE = 128
GAUSS_END = 8.0
BA = 128
NBLK = ATP // BA
E = ATP * NBR

SC_NC = 2
SC_NS = 16
SC_NW = SC_NC * SC_NS
EPW = E // SC_NW
CH = 128
NCHUNK = EPW // CH
NBUF = 5
LK = 3
L = 16


def _sc_gather_rows(table, idx):
    V, D = table.shape
    dt = table.dtype
    mesh = plsc.VectorSubcoreMesh(core_axis_name="c", subcore_axis_name="s")

    @functools.partial(
        pl.kernel,
        out_type=jax.ShapeDtypeStruct((E, D), dt),
        mesh=mesh,
        scratch_types=[
            pltpu.VMEM((EPW,), jnp.int32),
            pltpu.VMEM((NBUF, CH, D), dt),
        ] + [pltpu.SemaphoreType.DMA] * (2 * NBUF),
    )
    def gather_k(table_hbm, idx_hbm, out_hbm, idx_v, bufs, *sems):
        gsem = sems[:NBUF]
        wsem = sems[NBUF:]
        wid = lax.axis_index("s") * SC_NC + lax.axis_index("c")
        base = pl.multiple_of(wid * EPW, 8)
        pltpu.sync_copy(idx_hbm.at[pl.ds(base, EPW)], idx_v)

        def gcopy(c, b):
            off = pl.multiple_of(c * CH, 8)
            return pltpu.make_async_copy(
                table_hbm.at[idx_v.at[pl.ds(off, CH)]], bufs.at[b], gsem[b])

        def wcopy(c, b):
            off = pl.multiple_of(base + c * CH, 8)
            return pltpu.make_async_copy(
                bufs.at[b], out_hbm.at[pl.ds(off, CH)], wsem[b])

        for c in range(LK):
            gcopy(c, c).start()

        def body(p, carry):
            c0 = NBUF * p
            for b in range(NBUF):
                c = c0 + b
                gcopy(c, b).wait()
                wcopy(c, b).start()

                @pl.when(c + LK < NCHUNK)
                def _():
                    @pl.when(c + LK >= NBUF)
                    def _():
                        wcopy(c + LK - NBUF, (b + LK) % NBUF).wait()
                    gcopy(c + LK, (b + LK) % NBUF).start()
            return carry

        lax.fori_loop(0, NCHUNK // NBUF, body, 0)
        for c in range(NCHUNK - NBUF, NCHUNK):
            wcopy(c, c % NBUF).wait()

    return gather_k(table, idx)


def _sc_gather_coords(px, py, pz, an, idx):
    mesh = plsc.VectorSubcoreMesh(core_axis_name="c", subcore_axis_name="s")
    shp = jax.ShapeDtypeStruct((E,), jnp.float32)
    shpi = jax.ShapeDtypeStruct((E,), jnp.int32)

    @functools.partial(
        pl.kernel,
        out_type=[shp, shp, shp, shpi],
        mesh=mesh,
        compiler_params=pltpu.CompilerParams(needs_layout_passes=False),
        scratch_types=[
            pltpu.VMEM((ATP,), jnp.float32),
            pltpu.VMEM((ATP,), jnp.float32),
            pltpu.VMEM((ATP,), jnp.float32),
            pltpu.VMEM((ATP,), jnp.int32),
            pltpu.VMEM((EPW,), jnp.int32),
            pltpu.VMEM((EPW,), jnp.float32),
            pltpu.VMEM((EPW,), jnp.float32),
            pltpu.VMEM((EPW,), jnp.float32),
            pltpu.VMEM((EPW,), jnp.int32),
        ],
    )
    def coords_k(px_hbm, py_hbm, pz_hbm, an_hbm, idx_hbm,
                 ox_hbm, oy_hbm, oz_hbm, oa_hbm,
                 px_v, py_v, pz_v, an_v, idx_v, bx_v, by_v, bz_v, ba_v):
        wid = lax.axis_index("s") * SC_NC + lax.axis_index("c")
        base = pl.multiple_of(wid * EPW, 8)
        pltpu.sync_copy(px_hbm, px_v)
        pltpu.sync_copy(py_hbm, py_v)
        pltpu.sync_copy(pz_hbm, pz_v)
        pltpu.sync_copy(an_hbm, an_v)
        pltpu.sync_copy(idx_hbm.at[pl.ds(base, EPW)], idx_v)

        def body(g, carry):
            off = pl.multiple_of(g * L, 8)
            idxj = idx_v[pl.ds(off, L)]
            bx_v[pl.ds(off, L)] = plsc.load_gather(px_v, [idxj])
            by_v[pl.ds(off, L)] = plsc.load_gather(py_v, [idxj])
            bz_v[pl.ds(off, L)] = plsc.load_gather(pz_v, [idxj])
            ba_v[pl.ds(off, L)] = plsc.load_gather(an_v, [idxj])
            return carry

        lax.fori_loop(0, EPW // L, body, 0)
        pltpu.sync_copy(bx_v, ox_hbm.at[pl.ds(base, EPW)])
        pltpu.sync_copy(by_v, oy_hbm.at[pl.ds(base, EPW)])
        pltpu.sync_copy(bz_v, oz_hbm.at[pl.ds(base, EPW)])
        pltpu.sync_copy(ba_v, oa_hbm.at[pl.ds(base, EPW)])

    return coords_k(px, py, pz, an, idx)


def _geom_body(pos_ref, xj_ref, yj_ref, zj_ref, edge_ref, unit_ref):
    pos = pos_ref[...]
    rx = xj_ref[...] - pos[:, 0:1]
    ry = yj_ref[...] - pos[:, 1:2]
    rz = zj_ref[...] - pos[:, 2:3]
    dist = jnp.sqrt(rx * rx + ry * ry + rz * rz + 1e-10)
    inv = 1.0 / dist
    lane4 = lax.broadcasted_iota(jnp.int32, (BA, NBR, 4), 2)
    unit_ref[...] = jnp.where(
        lane4 == 0, (rx * inv)[:, :, None],
        jnp.where(lane4 == 1, (ry * inv)[:, :, None],
                  jnp.where(lane4 == 2, (rz * inv)[:, :, None], 0.0)))
    centers = lax.broadcasted_iota(jnp.int32, (BA, NBR, FE), 2).astype(
        jnp.float32) * (GAUSS_END / (FE - 1))
    z = (dist[:, :, None] - centers) * ((FE - 1) / GAUSS_END)
    edge_ref[...] = jnp.exp(-0.5 * z * z)


def _layer0_body(an_ref, anj_ref, edge_ref, emb_ref, wself_ref, wnbh_ref,
                 wedge_ref, bias_ref, wnbh1_ref, node_o_ref, edge_o_ref,
                 proj_o_ref):
    emb = emb_ref[...]
    an = an_ref[...]
    onehot_i = (an == lax.broadcasted_iota(jnp.int32, (BA, 128), 1)
                ).astype(jnp.float32)
    nodeb = jnp.dot(onehot_i, emb, preferred_element_type=jnp.float32)
    self_proj = jnp.dot(nodeb, wself_ref[...],
                        preferred_element_type=jnp.float32) + bias_ref[...]
    embw = jnp.dot(emb, wnbh_ref[...],
                   preferred_element_type=jnp.float32)
    anj3 = anj_ref[...][:, :, None]
    onehot_j = (anj3 == lax.broadcasted_iota(jnp.int32, (BA, NBR, 128), 2)
                ).astype(jnp.float32).reshape(BA * NBR, 128)
    edg = edge_ref[...].reshape(BA * NBR, FE)
    u = (jnp.dot(onehot_j, embw, preferred_element_type=jnp.float32)
         + jnp.dot(edg, wedge_ref[...], preferred_element_type=jnp.float32))
    u3 = u.reshape(BA, NBR, 3 * F) + self_proj[:, None, :]
    un = u3[..., 0:F]
    ug = u3[..., F:2 * F]
    ue = u3[..., 2 * F:3 * F]
    msg = jnp.tanh(un) * (1.0 / (1.0 + jnp.exp(-ug)))
    node1 = nodeb + jnp.sum(msg, axis=1)
    node_o_ref[...] = node1
    edge_o_ref[...] = edge_ref[...] + jnp.tanh(ue)
    proj_o_ref[...] = jnp.dot(node1, wnbh1_ref[...],
                              preferred_element_type=jnp.float32)


def _layer1_body(node_ref, nbhp_ref, edge_ref, unit_ref, wself_ref,
                 wedge_ref, be_ref, wo1_ref, bo1_ref, wo2_ref, bo2_ref,
                 out_ref):
    self_proj = jnp.dot(node_ref[...], wself_ref[...],
                        preferred_element_type=jnp.float32) + be_ref[...]
    edg = edge_ref[...].reshape(BA * NBR, FE)
    u = (nbhp_ref[...].reshape(BA * NBR, FE)
         + jnp.dot(edg, wedge_ref[...], preferred_element_type=jnp.float32))
    u3 = u.reshape(BA, NBR, FE) + self_proj[:, None, :]
    e2 = edge_ref[...] + jnp.tanh(u3)
    h = jnp.dot(e2.reshape(BA * NBR, FE), wo1_ref[...],
                preferred_element_type=jnp.float32) + bo1_ref[...]
    hs = jnp.maximum(h, 0.0) + jnp.log(1.0 + jnp.exp(-jnp.abs(h)))
    fm = jnp.sum(hs.reshape(BA, NBR, FE // 2) * wo2_ref[...][:, None, :],
                 axis=-1) + bo2_ref[0]
    out_ref[...] = jnp.sum(fm[:, :, None] * unit_ref[...], axis=1)


def _full(shape):
    return pl.BlockSpec(shape, lambda i: tuple(0 for _ in shape))


def kernel(positions, atomic_numbers, neighbors, emb, Wn, bn, Wgn, bgn,
           We, be, Wo1, bo1, Wo2, bo2):
    pos = jnp.pad(positions[0], ((0, ATP - AT), (0, 0)))
    pos16 = jnp.pad(pos, ((0, 0), (0, 13)))
    an = jnp.pad(atomic_numbers[0].astype(jnp.int32),
                 (0, ATP - AT)).reshape(ATP, 1)
    pad_idx = jnp.arange(E - AT * NBR, dtype=jnp.int32) % AT
    nbr = jnp.concatenate(
        [neighbors[0].astype(jnp.int32).reshape(AT * NBR), pad_idx])
    emb_pad = jnp.pad(emb, ((0, 128 - emb.shape[0]), (0, 0)))

    def third(k):
        return jnp.concatenate(
            [Wn[0][k * F:(k + 1) * F], Wgn[0][k * F:(k + 1) * F],
             We[0][k * F:(k + 1) * F]], axis=1)
    wself0, wnbh0, wedge0 = third(0), third(1), third(2)
    bias0 = jnp.concatenate([bn[0], bgn[0], be[0]]).reshape(1, 3 * F)
    wself1 = We[1][0:F]
    wnbh1 = We[1][F:2 * F]
    wedge1 = We[1][2 * F:3 * F]
    bias1 = be[1].reshape(1, FE)

    xj, yj, zj, anj = _sc_gather_coords(
        pos[:, 0], pos[:, 1], pos[:, 2], an.reshape(ATP), nbr)
    xj = xj.reshape(ATP, NBR)
    yj = yj.reshape(ATP, NBR)
    zj = zj.reshape(ATP, NBR)
    anj = anj.reshape(ATP, NBR)

    edge0, unit = pl.pallas_call(
        _geom_body,
        grid=(NBLK,),
        in_specs=[
            pl.BlockSpec((BA, 16), lambda i: (i, 0)),
            pl.BlockSpec((BA, NBR), lambda i: (i, 0)),
            pl.BlockSpec((BA, NBR), lambda i: (i, 0)),
            pl.BlockSpec((BA, NBR), lambda i: (i, 0)),
        ],
        out_specs=[
            pl.BlockSpec((BA, NBR, FE), lambda i: (i, 0, 0)),
            pl.BlockSpec((BA, NBR, 4), lambda i: (i, 0, 0)),
        ],
        out_shape=[
            jax.ShapeDtypeStruct((ATP, NBR, FE), jnp.float32),
            jax.ShapeDtypeStruct((ATP, NBR, 4), jnp.float32),
        ],
    )(pos16, xj, yj, zj)

    node1, edge1, proj1 = pl.pallas_call(
        _layer0_body,
        grid=(NBLK,),
        in_specs=[
            pl.BlockSpec((BA, 1), lambda i: (i, 0)),
            pl.BlockSpec((BA, NBR), lambda i: (i, 0)),
            pl.BlockSpec((BA, NBR, FE), lambda i: (i, 0, 0)),
            _full((128, F)),
            _full((F, 3 * F)),
            _full((F, 3 * F)),
            _full((FE, 3 * F)),
            _full((1, 3 * F)),
            _full((F, FE)),
        ],
        out_specs=[
            pl.BlockSpec((BA, F), lambda i: (i, 0)),
            pl.BlockSpec((BA, NBR, FE), lambda i: (i, 0, 0)),
            pl.BlockSpec((BA, FE), lambda i: (i, 0)),
        ],
        out_shape=[
            jax.ShapeDtypeStruct((ATP, F), jnp.float32),
            jax.ShapeDtypeStruct((ATP, NBR, FE), jnp.float32),
            jax.ShapeDtypeStruct((ATP, FE), jnp.float32),
        ],
    )(an, anj, edge0, emb_pad, wself0, wnbh0, wedge0, bias0, wnbh1)

    nbhp1 = _sc_gather_rows(proj1, nbr).reshape(ATP, NBR, FE)

    forces = pl.pallas_call(
        _layer1_body,
        grid=(NBLK,),
        in_specs=[
            pl.BlockSpec((BA, F), lambda i: (i, 0)),
            pl.BlockSpec((BA, NBR, FE), lambda i: (i, 0, 0)),
            pl.BlockSpec((BA, NBR, FE), lambda i: (i, 0, 0)),
            pl.BlockSpec((BA, NBR, 4), lambda i: (i, 0, 0)),
            _full((F, FE)),
            _full((FE, FE)),
            _full((1, FE)),
            _full((FE, FE // 2)),
            _full((1, FE // 2)),
            _full((1, FE // 2)),
            pl.BlockSpec(memory_space=pltpu.SMEM),
        ],
        out_specs=[pl.BlockSpec((BA, 4), lambda i: (i, 0))],
        out_shape=[jax.ShapeDtypeStruct((ATP, 4), jnp.float32)],
    )(node1, nbhp1, edge1, unit, wself1, wedge1, bias1,
      Wo1, bo1.reshape(1, FE // 2), Wo2.reshape(1, FE // 2), bo2)[0]

    return forces[0:AT, 0:3].reshape(1, AT, 3)


# speedup vs baseline: 18.0223x; 1.1426x over previous
import functools

import jax
import jax.numpy as jnp
from jax import lax
from jax.experimental import pallas as pl
from jax.experimental.pallas import tpu as pltpu
from jax.experimental.pallas import tpu_sc as plsc

AT = 10000
ATP = 10240
NBR = 32
F = 128
FE = 128
GAUSS_END = 8.0
BA = 128
NBLK = ATP // BA
E = ATP * NBR

SC_NC = 2
SC_NS = 16
SC_NW = SC_NC * SC_NS
EPW = E // SC_NW
CH = 128
NCHUNK = EPW // CH
NBUF = 5
LK = 3
L = 16


def _sc_gather_rows(table, idx):
    V, D = table.shape
    dt = table.dtype
    mesh = plsc.VectorSubcoreMesh(core_axis_name="c", subcore_axis_name="s")

    @functools.partial(
        pl.kernel,
        out_type=jax.ShapeDtypeStruct((E, D), dt),
        mesh=mesh,
        scratch_types=[
            pltpu.VMEM((EPW,), jnp.int32),
            pltpu.VMEM((NBUF, CH, D), dt),
        ] + [pltpu.SemaphoreType.DMA] * (2 * NBUF),
    )
    def gather_k(table_hbm, idx_hbm, out_hbm, idx_v, bufs, *sems):
        gsem = sems[:NBUF]
        wsem = sems[NBUF:]
        wid = lax.axis_index("s") * SC_NC + lax.axis_index("c")
        base = pl.multiple_of(wid * EPW, 8)
        pltpu.sync_copy(idx_hbm.at[pl.ds(base, EPW)], idx_v)

        def gcopy(c, b):
            off = pl.multiple_of(c * CH, 8)
            return pltpu.make_async_copy(
                table_hbm.at[idx_v.at[pl.ds(off, CH)]], bufs.at[b], gsem[b])

        def wcopy(c, b):
            off = pl.multiple_of(base + c * CH, 8)
            return pltpu.make_async_copy(
                bufs.at[b], out_hbm.at[pl.ds(off, CH)], wsem[b])

        for c in range(LK):
            gcopy(c, c).start()

        def body(p, carry):
            c0 = NBUF * p
            for b in range(NBUF):
                c = c0 + b
                gcopy(c, b).wait()
                wcopy(c, b).start()

                @pl.when(c + LK < NCHUNK)
                def _():
                    @pl.when(c + LK >= NBUF)
                    def _():
                        wcopy(c + LK - NBUF, (b + LK) % NBUF).wait()
                    gcopy(c + LK, (b + LK) % NBUF).start()
            return carry

        lax.fori_loop(0, NCHUNK // NBUF, body, 0)
        for c in range(NCHUNK - NBUF, NCHUNK):
            wcopy(c, c % NBUF).wait()

    return gather_k(table, idx)


def _sc_gather_coords(px, py, pz, an, idx):
    mesh = plsc.VectorSubcoreMesh(core_axis_name="c", subcore_axis_name="s")
    shp = jax.ShapeDtypeStruct((E,), jnp.float32)
    shpi = jax.ShapeDtypeStruct((E,), jnp.int32)

    @functools.partial(
        pl.kernel,
        out_type=[shp, shp, shp, shpi],
        mesh=mesh,
        compiler_params=pltpu.CompilerParams(needs_layout_passes=False),
        scratch_types=[
            pltpu.VMEM((ATP,), jnp.float32),
            pltpu.VMEM((ATP,), jnp.float32),
            pltpu.VMEM((ATP,), jnp.float32),
            pltpu.VMEM((ATP,), jnp.int32),
            pltpu.VMEM((EPW,), jnp.int32),
            pltpu.VMEM((EPW,), jnp.float32),
            pltpu.VMEM((EPW,), jnp.float32),
            pltpu.VMEM((EPW,), jnp.float32),
            pltpu.VMEM((EPW,), jnp.int32),
        ],
    )
    def coords_k(px_hbm, py_hbm, pz_hbm, an_hbm, idx_hbm,
                 ox_hbm, oy_hbm, oz_hbm, oa_hbm,
                 px_v, py_v, pz_v, an_v, idx_v, bx_v, by_v, bz_v, ba_v):
        wid = lax.axis_index("s") * SC_NC + lax.axis_index("c")
        base = pl.multiple_of(wid * EPW, 8)
        pltpu.sync_copy(px_hbm, px_v)
        pltpu.sync_copy(py_hbm, py_v)
        pltpu.sync_copy(pz_hbm, pz_v)
        pltpu.sync_copy(an_hbm, an_v)
        pltpu.sync_copy(idx_hbm.at[pl.ds(base, EPW)], idx_v)

        def body(g, carry):
            off = pl.multiple_of(g * L, 8)
            idxj = idx_v[pl.ds(off, L)]
            bx_v[pl.ds(off, L)] = plsc.load_gather(px_v, [idxj])
            by_v[pl.ds(off, L)] = plsc.load_gather(py_v, [idxj])
            bz_v[pl.ds(off, L)] = plsc.load_gather(pz_v, [idxj])
            ba_v[pl.ds(off, L)] = plsc.load_gather(an_v, [idxj])
            return carry

        lax.fori_loop(0, EPW // L, body, 0)
        pltpu.sync_copy(bx_v, ox_hbm.at[pl.ds(base, EPW)])
        pltpu.sync_copy(by_v, oy_hbm.at[pl.ds(base, EPW)])
        pltpu.sync_copy(bz_v, oz_hbm.at[pl.ds(base, EPW)])
        pltpu.sync_copy(ba_v, oa_hbm.at[pl.ds(base, EPW)])

    return coords_k(px, py, pz, an, idx)


def _layer0_body(an_ref, anj_ref, pos_ref, xj_ref, yj_ref, zj_ref, emb_ref,
                 wself_ref, wnbh_ref, wedge_ref, bias_ref, wnbh1_ref,
                 wself1_ref, be1_ref, edge_o_ref, proj_o_ref, selfp_o_ref,
                 unit_ref):
    pos = pos_ref[...]
    rx = xj_ref[...] - pos[:, 0:1]
    ry = yj_ref[...] - pos[:, 1:2]
    rz = zj_ref[...] - pos[:, 2:3]
    dist = jnp.sqrt(rx * rx + ry * ry + rz * rz + 1e-10)
    inv = 1.0 / dist
    lane4 = lax.broadcasted_iota(jnp.int32, (BA, NBR, 4), 2)
    unit_ref[...] = jnp.where(
        lane4 == 0, (rx * inv)[:, :, None],
        jnp.where(lane4 == 1, (ry * inv)[:, :, None],
                  jnp.where(lane4 == 2, (rz * inv)[:, :, None], 0.0)))
    centers = lax.broadcasted_iota(jnp.int32, (BA, NBR, FE), 2).astype(
        jnp.float32) * (GAUSS_END / (FE - 1))
    z = (dist[:, :, None] - centers) * ((FE - 1) / GAUSS_END)
    edge0 = jnp.exp(-0.5 * z * z)
    emb = emb_ref[...]
    an = an_ref[...]
    onehot_i = (an == lax.broadcasted_iota(jnp.int32, (BA, 128), 1)
                ).astype(jnp.float32)
    nodeb = jnp.dot(onehot_i, emb, preferred_element_type=jnp.float32)
    self_proj = jnp.dot(nodeb, wself_ref[...],
                        preferred_element_type=jnp.float32) + bias_ref[...]
    embw = jnp.dot(emb, wnbh_ref[...],
                   preferred_element_type=jnp.float32)
    anj3 = anj_ref[...][:, :, None]
    onehot_j = (anj3 == lax.broadcasted_iota(jnp.int32, (BA, NBR, 128), 2)
                ).astype(jnp.float32).reshape(BA * NBR, 128)
    edg = edge0.reshape(BA * NBR, FE)
    u = (jnp.dot(onehot_j, embw, preferred_element_type=jnp.float32)
         + jnp.dot(edg, wedge_ref[...], preferred_element_type=jnp.float32))
    u3 = u.reshape(BA, NBR, 3 * F) + self_proj[:, None, :]
    un = u3[..., 0:F]
    ug = u3[..., F:2 * F]
    ue = u3[..., 2 * F:3 * F]
    msg = jnp.tanh(un) * (1.0 / (1.0 + jnp.exp(-ug)))
    node1 = nodeb + jnp.sum(msg, axis=1)
    edge_o_ref[...] = edge0 + jnp.tanh(ue)
    proj_o_ref[...] = jnp.dot(node1, wnbh1_ref[...],
                              preferred_element_type=jnp.float32)
    selfp_o_ref[...] = jnp.dot(node1, wself1_ref[...],
                               preferred_element_type=jnp.float32) + be1_ref[...]


def _layer1_body(selfp_ref, nbhp_ref, edge_ref, unit_ref, wedge_ref,
                 wo1_ref, bo1_ref, wo2_ref, bo2_ref, out_ref):
    edg = edge_ref[...].reshape(BA * NBR, FE)
    u = (nbhp_ref[...].reshape(BA * NBR, FE)
         + jnp.dot(edg, wedge_ref[...], preferred_element_type=jnp.float32))
    u3 = u.reshape(BA, NBR, FE) + selfp_ref[...][:, None, :]
    e2 = edge_ref[...] + jnp.tanh(u3)
    h = jnp.dot(e2.reshape(BA * NBR, FE), wo1_ref[...],
                preferred_element_type=jnp.float32) + bo1_ref[...]
    hs = jnp.maximum(h, 0.0) + jnp.log(1.0 + jnp.exp(-jnp.abs(h)))
    fm = jnp.sum(hs.reshape(BA, NBR, FE // 2) * wo2_ref[...][:, None, :],
                 axis=-1) + bo2_ref[0]
    out_ref[...] = jnp.sum(fm[:, :, None] * unit_ref[...], axis=1)


def _full(shape):
    return pl.BlockSpec(shape, lambda i: tuple(0 for _ in shape))


def kernel(positions, atomic_numbers, neighbors, emb, Wn, bn, Wgn, bgn,
           We, be, Wo1, bo1, Wo2, bo2):
    pos = jnp.pad(positions[0], ((0, ATP - AT), (0, 0)))
    pos16 = jnp.pad(pos, ((0, 0), (0, 13)))
    an = jnp.pad(atomic_numbers[0].astype(jnp.int32),
                 (0, ATP - AT)).reshape(ATP, 1)
    pad_idx = jnp.arange(E - AT * NBR, dtype=jnp.int32) % AT
    nbr = jnp.concatenate(
        [neighbors[0].astype(jnp.int32).reshape(AT * NBR), pad_idx])
    emb_pad = jnp.pad(emb, ((0, 128 - emb.shape[0]), (0, 0)))

    def third(k):
        return jnp.concatenate(
            [Wn[0][k * F:(k + 1) * F], Wgn[0][k * F:(k + 1) * F],
             We[0][k * F:(k + 1) * F]], axis=1)
    wself0, wnbh0, wedge0 = third(0), third(1), third(2)
    bias0 = jnp.concatenate([bn[0], bgn[0], be[0]]).reshape(1, 3 * F)
    wself1 = We[1][0:F]
    wnbh1 = We[1][F:2 * F]
    wedge1 = We[1][2 * F:3 * F]
    bias1 = be[1].reshape(1, FE)

    xj, yj, zj, anj = _sc_gather_coords(
        pos[:, 0], pos[:, 1], pos[:, 2], an.reshape(ATP), nbr)
    xj = xj.reshape(ATP, NBR)
    yj = yj.reshape(ATP, NBR)
    zj = zj.reshape(ATP, NBR)
    anj = anj.reshape(ATP, NBR)

    edge1, proj1, selfp1, unit = pl.pallas_call(
        _layer0_body,
        grid=(NBLK,),
        in_specs=[
            pl.BlockSpec((BA, 1), lambda i: (i, 0)),
            pl.BlockSpec((BA, NBR), lambda i: (i, 0)),
            pl.BlockSpec((BA, 16), lambda i: (i, 0)),
            pl.BlockSpec((BA, NBR), lambda i: (i, 0)),
            pl.BlockSpec((BA, NBR), lambda i: (i, 0)),
            pl.BlockSpec((BA, NBR), lambda i: (i, 0)),
            _full((128, F)),
            _full((F, 3 * F)),
            _full((F, 3 * F)),
            _full((FE, 3 * F)),
            _full((1, 3 * F)),
            _full((F, FE)),
            _full((F, FE)),
            _full((1, FE)),
        ],
        out_specs=[
            pl.BlockSpec((BA, NBR, FE), lambda i: (i, 0, 0)),
            pl.BlockSpec((BA, FE), lambda i: (i, 0)),
            pl.BlockSpec((BA, FE), lambda i: (i, 0)),
            pl.BlockSpec((BA, NBR, 4), lambda i: (i, 0, 0)),
        ],
        out_shape=[
            jax.ShapeDtypeStruct((ATP, NBR, FE), jnp.float32),
            jax.ShapeDtypeStruct((ATP, FE), jnp.float32),
            jax.ShapeDtypeStruct((ATP, FE), jnp.float32),
            jax.ShapeDtypeStruct((ATP, NBR, 4), jnp.float32),
        ],
    )(an, anj, pos16, xj, yj, zj, emb_pad, wself0, wnbh0, wedge0, bias0,
      wnbh1, wself1, bias1)

    nbhp1 = _sc_gather_rows(proj1, nbr).reshape(ATP, NBR, FE)

    forces = pl.pallas_call(
        _layer1_body,
        grid=(NBLK,),
        in_specs=[
            pl.BlockSpec((BA, FE), lambda i: (i, 0)),
            pl.BlockSpec((BA, NBR, FE), lambda i: (i, 0, 0)),
            pl.BlockSpec((BA, NBR, FE), lambda i: (i, 0, 0)),
            pl.BlockSpec((BA, NBR, 4), lambda i: (i, 0, 0)),
            _full((FE, FE)),
            _full((FE, FE // 2)),
            _full((1, FE // 2)),
            _full((1, FE // 2)),
            pl.BlockSpec(memory_space=pltpu.SMEM),
        ],
        out_specs=[pl.BlockSpec((BA, 4), lambda i: (i, 0))],
        out_shape=[jax.ShapeDtypeStruct((ATP, 4), jnp.float32)],
    )(selfp1, nbhp1, edge1, unit, wedge1,
      Wo1, bo1.reshape(1, FE // 2), Wo2.reshape(1, FE // 2), bo2)[0]

    return forces[0:AT, 0:3].reshape(1, AT, 3)
